# Initial kernel scaffold; baseline (speedup 1.0000x reference)
import functools

import jax
import jax.numpy as jnp
from jax import lax
from jax.experimental import pallas as pl
from jax.experimental.pallas import tpu as pltpu
from jax.experimental.pallas import tpu_sc as plsc

NC = 2
NS = 16
LANES = 16
C = 128
BLK = 64


def _make_spmm(num_chunks: int, n: int, ept: int):
    assert num_chunks % NC == 0 and ept % BLK == 0 and n % NS == 0
    chunks_per_core = num_chunks // NC
    nblk = ept // BLK
    rows_per_tile = n // NS
    FL = 125
    assert rows_per_tile % FL == 0
    nfl = rows_per_tile // FL

    mesh = plsc.VectorSubcoreMesh(
        core_axis_name="c", subcore_axis_name="s",
        num_cores=NC, num_subcores=NS)

    @functools.partial(
        pl.kernel,
        out_type=jax.ShapeDtypeStruct((num_chunks, n, C), jnp.float32),
        mesh=mesh,
        scratch_types=[
            pltpu.VMEM((ept,), jnp.int32),
            pltpu.VMEM((ept,), jnp.int32),
            pltpu.VMEM((ept,), jnp.float32),
            pltpu.VMEM((BLK, C), jnp.float32),
            pltpu.VMEM((BLK,), jnp.int32),
            pltpu.VMEM((BLK,), jnp.int32),
            pltpu.VMEM((FL, C), jnp.float32),
            pltpu.VMEM_SHARED((n, C), jnp.float32),
            pltpu.SemaphoreType.DMA,
        ],
    )
    def spmm(h_hbm, cols_hbm, rows_hbm, vals_hbm, out_hbm,
             cols_v, rows_v, vals_v, gbuf, idx_g, idx_s, zbuf, acc, gsem):
        cid = lax.axis_index("c")
        sid = lax.axis_index("s")
        ebase = sid * ept
        pltpu.sync_copy(cols_hbm.at[pl.ds(ebase, ept)], cols_v)
        pltpu.sync_copy(rows_hbm.at[pl.ds(ebase, ept)], rows_v)
        pltpu.sync_copy(vals_hbm.at[pl.ds(ebase, ept)], vals_v)

        zeros16 = jnp.zeros((LANES,), jnp.float32)

        def zrow(r, carry):
            for c8 in range(C // LANES):
                zbuf[r, pl.ds(c8 * LANES, LANES)] = zeros16
            return carry
        lax.fori_loop(0, FL, zrow, 0)

        for jj in range(chunks_per_core):
            j = cid + jj * NC
            for p in range(nfl):
                r0 = sid * rows_per_tile + p * FL
                pltpu.sync_copy(zbuf, acc.at[pl.ds(r0, FL)])
            plsc.subcore_barrier()

            joff = j * n

            def eblk(blk, carry):
                eoff = blk * BLK
                for q in range(BLK // LANES):
                    cv = cols_v[pl.ds(eoff + q * LANES, LANES)]
                    idx_g[pl.ds(q * LANES, LANES)] = cv + joff
                    idx_s[pl.ds(q * LANES, LANES)] = (
                        rows_v[pl.ds(eoff + q * LANES, LANES)])
                pltpu.async_copy(h_hbm.at[idx_g], gbuf, gsem).wait()

                def scale(b, inner):
                    valv = plsc.load_gather(
                        vals_v, [jnp.full((LANES,), eoff + b, jnp.int32)])
                    for k in range(C // LANES):
                        g = gbuf[b, pl.ds(k * LANES, LANES)]
                        gbuf[b, pl.ds(k * LANES, LANES)] = g * valv
                    return inner
                lax.fori_loop(0, BLK, scale, 0)
                pltpu.sync_copy(gbuf, acc.at[idx_s], add=True)
                return carry
            lax.fori_loop(0, nblk, eblk, 0)
            plsc.subcore_barrier()

            for p in range(nfl):
                r0 = sid * rows_per_tile + p * FL
                pltpu.sync_copy(acc.at[pl.ds(r0, FL)],
                                out_hbm.at[j].at[pl.ds(r0, FL)])
            plsc.subcore_barrier()

    return spmm


def _gemm_l0(t_num: int, n: int, d: int, bn: int):
    kin = d // C
    kout = t_num * d // C

    def body(s_ref, w_ref, out_ref):
        x = jnp.concatenate([s_ref[i] for i in range(kin)], axis=1)
        for t in range(t_num):
            y = lax.dot(x, w_ref[t], precision=lax.Precision.HIGHEST,
                        preferred_element_type=jnp.float32)
            y = jnp.maximum(y, 0.0)
            for u in range(kin):
                out_ref[t * kin + u] = y[:, u * C:(u + 1) * C]

    return pl.pallas_call(
        body,
        grid=(n // bn,),
        in_specs=[
            pl.BlockSpec((kin, bn, C), lambda i: (0, i, 0)),
            pl.BlockSpec((t_num, d, d), lambda i: (0, 0, 0)),
        ],
        out_specs=pl.BlockSpec((kout, bn, C), lambda i: (0, i, 0)),
        out_shape=jax.ShapeDtypeStruct((kout, n, C), jnp.float32),
    )


def _gemm_l1(t_num: int, n: int, d: int, bn: int):
    kin = d // C

    def body(s_ref, w_ref, et_ref, out_ref):
        et = et_ref[...]
        accum = jnp.zeros((bn, d), jnp.float32)
        for t in range(t_num):
            xt = jnp.concatenate(
                [s_ref[t * kin + u] for u in range(kin)], axis=1)
            y = lax.dot(xt, w_ref[t], precision=lax.Precision.HIGHEST,
                        preferred_element_type=jnp.float32)
            y = jnp.maximum(y, 0.0)
            accum = jnp.where(et == t, y, accum)
        out_ref[...] = accum

    return pl.pallas_call(
        body,
        grid=(n // bn,),
        in_specs=[
            pl.BlockSpec((t_num * kin, bn, C), lambda i: (0, i, 0)),
            pl.BlockSpec((t_num, d, d), lambda i: (0, 0, 0)),
            pl.BlockSpec((bn, 1), lambda i: (i, 0)),
        ],
        out_specs=pl.BlockSpec((bn, d), lambda i: (i, 0)),
        out_shape=jax.ShapeDtypeStruct((n, d), jnp.float32),
    )


def kernel(features, edge_types, adj_rows, adj_cols, adj_vals, W):
    n, d = features.shape
    e = adj_rows.shape[0]
    t_num = W.shape[0]
    bn = 1000

    ept = -(-e // (NS * BLK)) * BLK
    epad = NS * ept - e
    cols_p = jnp.pad(adj_cols.astype(jnp.int32), (0, epad))
    rows_p = jnp.pad(adj_rows.astype(jnp.int32), (0, epad))
    vals_p = jnp.pad(adj_vals, (0, epad))

    f_flat = features.reshape(n, d // C, C).transpose(1, 0, 2).reshape(-1, C)
    s0 = _make_spmm(d // C, n, ept)(f_flat, cols_p, rows_p, vals_p)

    w0t = jnp.transpose(W[:, 0], (0, 2, 1))
    h1 = _gemm_l0(t_num, n, d, bn)(s0, w0t)

    s1 = _make_spmm(t_num * d // C, n, ept)(
        h1.reshape(-1, C), cols_p, rows_p, vals_p)

    w1t = jnp.transpose(W[:, 1], (0, 2, 1))
    et2 = edge_types.reshape(n, 1).astype(jnp.int32)
    return _gemm_l1(t_num, n, d, bn)(s1, w1t, et2)


# speedup vs baseline: 3.9236x; 3.9236x over previous
import functools

import jax
import jax.numpy as jnp
from jax import lax
from jax.experimental import pallas as pl
from jax.experimental.pallas import tpu as pltpu
from jax.experimental.pallas import tpu_sc as plsc

NC = 2
NS = 16
LANES = 16
C = 64
BLK = 128


def _make_spmm(num_chunks: int, n: int, ept: int,
               spmem_table: bool = False):
    assert num_chunks % NC == 0 and ept % (3 * BLK) == 0 and n % 8 == 0
    chunks_per_core = num_chunks // NC
    nblk = ept // BLK
    RPT = (n // 8 // NS) * 8
    REM = n - NS * RPT
    FL = 208
    assert RPT % FL == 0 and REM % 8 == 0 and REM < FL
    nfl = RPT // FL

    mesh = plsc.VectorSubcoreMesh(
        core_axis_name="c", subcore_axis_name="s",
        num_cores=NC, num_subcores=NS)

    @functools.partial(
        pl.kernel,
        out_type=jax.ShapeDtypeStruct((num_chunks, n, C), jnp.float32),
        mesh=mesh,
        compiler_params=pltpu.CompilerParams(
            use_tc_tiling_on_sc=False, needs_layout_passes=False),
        scratch_types=[
            pltpu.VMEM((ept,), jnp.int32),
            pltpu.VMEM((ept,), jnp.bfloat16),
            [pltpu.VMEM((BLK, C),
                        jnp.bfloat16 if spmem_table else jnp.float32)
             for _ in range(3)],
            ([pltpu.VMEM((BLK, C), jnp.float32) for _ in range(3)]
             if spmem_table else []),
            [pltpu.VMEM((BLK,), jnp.int32) for _ in range(3)],
            [pltpu.VMEM((BLK,), jnp.int32) for _ in range(3)],
            pltpu.VMEM((FL, C), jnp.float32),
            pltpu.VMEM_SHARED((n, C), jnp.float32),
            ([pltpu.VMEM_SHARED((n, C), jnp.bfloat16)]
             if spmem_table else []),
            [pltpu.SemaphoreType.DMA for _ in range(3)],
            [pltpu.SemaphoreType.DMA for _ in range(3)],
        ],
    )
    def spmm(h_hbm, rc_hbm, vals_hbm, out_hbm,
             rc_v, vals_v, gbuf, sbuf, idx_g, idx_s, zbuf, acc,
             tbl_l, gsem, ssem):
        tbl = tbl_l[0] if spmem_table else None
        src_of = (lambda p: sbuf[p]) if spmem_table else (lambda p: gbuf[p])
        cid = lax.axis_index("c")
        sid = lax.axis_index("s")
        ebase = sid * ept
        pltpu.sync_copy(rc_hbm.at[pl.ds(ebase, ept)], rc_v)
        pltpu.sync_copy(vals_hbm.at[pl.ds(ebase, ept)], vals_v)

        zeros16 = jnp.zeros((LANES,), jnp.float32)

        def zrow(r, carry):
            for c8 in range(C // LANES):
                zbuf[r, pl.ds(c8 * LANES, LANES)] = zeros16
            return carry
        lax.fori_loop(0, FL, zrow, 0)

        def build_idx(i, joff, p):
            eoff = i * BLK
            for q in range(BLK // LANES):
                rcv = rc_v[pl.ds(eoff + q * LANES, LANES)]
                idx_g[p][pl.ds(q * LANES, LANES)] = (rcv & 16383) + joff
                idx_s[p][pl.ds(q * LANES, LANES)] = (
                    lax.shift_right_logical(rcv, 14))

        def widen(xb):
            xi = plsc.bitcast(xb, jnp.int32)
            lo = plsc.bitcast(lax.shift_left(xi, 16), jnp.float32)
            hi = plsc.bitcast(
                lax.bitwise_and(xi, jnp.int32(-65536)), jnp.float32)
            return lo, hi

        def scale(i, p):
            eoff = i * BLK
            for g2 in range(BLK // (2 * LANES)):
                vb = vals_v[pl.ds(eoff + 2 * g2 * LANES, 2 * LANES)]
                vpair = widen(vb)
                for half in range(2):
                    vv = vpair[half]
                    for u in range(LANES):
                        valv = jnp.full((LANES,), vv[u], jnp.float32)
                        row = 2 * g2 * LANES + half * LANES + u
                        if not spmem_table:
                            for k in range(C // LANES):
                                g = gbuf[p][row, k * LANES:(k + 1) * LANES]
                                gbuf[p][row, k * LANES:(k + 1) * LANES] = (
                                    g * valv)
                        else:
                            for h in range(C // (2 * LANES)):
                                lo, hi = widen(
                                    gbuf[p][row,
                                            2 * h * LANES:2 * (h + 1) * LANES])
                                c0 = 2 * h * LANES
                                sbuf[p][row, c0:c0 + LANES] = lo * valv
                                sbuf[p][row, c0 + LANES:c0 + 2 * LANES] = (
                                    hi * valv)

        gsrc = tbl if spmem_table else h_hbm

        def gather_start(i, p):
            pltpu.make_async_copy(gsrc.at[idx_g[p]], gbuf[p], gsem[p]).start()

        def gather_wait(p):
            pltpu.make_async_copy(gsrc.at[idx_g[p]], gbuf[p], gsem[p]).wait()

        def scatter_start(p):
            pltpu.async_copy(src_of(p), acc.at[idx_s[p]], ssem[p], add=True)

        def scatter_wait(p):
            pltpu.make_async_copy(src_of(p), acc.at[idx_s[p]],
                                  ssem[p]).wait()

        def chunk_body(jj, carry):
            j = cid + jj * NC
            for p in range(nfl):
                r0 = sid * RPT + p * FL
                pltpu.sync_copy(zbuf, acc.at[pl.ds(r0, FL)])
            if REM:
                @pl.when(sid == NS - 1)
                def _zero_tail():
                    pltpu.sync_copy(zbuf.at[pl.ds(0, REM)],
                                    acc.at[pl.ds(NS * RPT, REM)])
            plsc.subcore_barrier()

            joff = jnp.int32(0) if spmem_table else j * n

            if spmem_table:
                t0 = sid * RPT
                pltpu.sync_copy(h_hbm.at[pl.ds(j * n + t0, RPT)],
                                tbl.at[pl.ds(t0, RPT)])
                if REM:
                    @pl.when(sid == NS - 1)
                    def _tbl_tail():
                        pltpu.sync_copy(
                            h_hbm.at[pl.ds(j * n + NS * RPT, REM)],
                            tbl.at[pl.ds(NS * RPT, REM)])
                plsc.subcore_barrier()

            build_idx(0, joff, 0)
            gather_start(0, 0)
            build_idx(1, joff, 1)
            gather_start(1, 1)

            def eblk3(g, carry):
                for p in range(3):
                    i = 3 * g + p
                    q = (p + 2) % 3
                    gather_wait(p)
                    scale(i, p)
                    scatter_start(p)

                    @pl.when(i >= 1)
                    def _drain_prev():
                        scatter_wait(q)

                    @pl.when(i + 2 < nblk)
                    def _prefetch():
                        build_idx(i + 2, joff, q)
                        gather_start(i + 2, q)
                return carry
            lax.fori_loop(0, nblk // 3, eblk3, 0)
            scatter_wait((nblk - 1) % 3)
            plsc.subcore_barrier()

            r0 = sid * RPT
            pltpu.sync_copy(acc.at[pl.ds(r0, RPT)],
                            out_hbm.at[j].at[pl.ds(r0, RPT)])
            if REM:
                @pl.when(sid == NS - 1)
                def _flush_tail():
                    pltpu.sync_copy(acc.at[pl.ds(NS * RPT, REM)],
                                    out_hbm.at[j].at[pl.ds(NS * RPT, REM)])
            plsc.subcore_barrier()
            return carry
        lax.fori_loop(0, chunks_per_core, chunk_body, 0)

    return spmm


def _gemm_l0(t_num: int, n: int, d: int, bn: int):
    kin = d // C
    kout = t_num * d // C

    def body(s_ref, w_ref, out_ref):
        x = jnp.concatenate([s_ref[i] for i in range(kin)], axis=1)
        for t in range(t_num):
            y = lax.dot(x, w_ref[t], precision=lax.Precision.HIGHEST,
                        preferred_element_type=jnp.float32)
            y = jnp.maximum(y, 0.0)
            for u in range(kin):
                out_ref[t * kin + u] = y[:, u * C:(u + 1) * C]

    return pl.pallas_call(
        body,
        grid=(n // bn,),
        in_specs=[
            pl.BlockSpec((kin, bn, C), lambda i: (0, i, 0)),
            pl.BlockSpec((t_num, d, d), lambda i: (0, 0, 0)),
        ],
        out_specs=pl.BlockSpec((kout, bn, C), lambda i: (0, i, 0)),
        out_shape=jax.ShapeDtypeStruct((kout, n, C), jnp.float32),
    )


def _gemm_l1(t_num: int, n: int, d: int, bn: int):
    kin = d // C

    def body(s_ref, w_ref, et_ref, out_ref):
        et = et_ref[...]
        accum = jnp.zeros((bn, d), jnp.float32)
        for t in range(t_num):
            xt = jnp.concatenate(
                [s_ref[t * kin + u] for u in range(kin)], axis=1)
            y = lax.dot(xt, w_ref[t], precision=lax.Precision.HIGHEST,
                        preferred_element_type=jnp.float32)
            y = jnp.maximum(y, 0.0)
            accum = jnp.where(et == t, y, accum)
        out_ref[...] = accum

    return pl.pallas_call(
        body,
        grid=(n // bn,),
        in_specs=[
            pl.BlockSpec((t_num * kin, bn, C), lambda i: (0, i, 0)),
            pl.BlockSpec((t_num, d, d), lambda i: (0, 0, 0)),
            pl.BlockSpec((bn, 1), lambda i: (i, 0)),
        ],
        out_specs=pl.BlockSpec((bn, d), lambda i: (i, 0)),
        out_shape=jax.ShapeDtypeStruct((n, d), jnp.float32),
    )


def kernel(features, edge_types, adj_rows, adj_cols, adj_vals, W):
    n, d = features.shape
    e = adj_rows.shape[0]
    t_num = W.shape[0]
    bn = 1000

    ept = -(-e // (NS * 3 * BLK)) * 3 * BLK
    epad = NS * ept - e
    rc_p = jnp.pad((adj_rows.astype(jnp.int32) << 14)
                   | adj_cols.astype(jnp.int32), (0, epad))
    vals_p = jnp.pad(adj_vals, (0, epad)).astype(jnp.bfloat16)
    vals_p = vals_p.reshape(-1, 2, 16).swapaxes(1, 2).reshape(-1)

    f_flat = features.reshape(n, d // C, C).transpose(1, 0, 2).reshape(-1, C)
    s0 = _make_spmm(d // C, n, ept)(f_flat, rc_p, vals_p)

    w0t = jnp.transpose(W[:, 0], (0, 2, 1))
    import numpy as _np
    perm = _np.arange(d).reshape(d // 32, 2, 16).transpose(0, 2, 1).reshape(-1)
    w0t = w0t[:, :, perm]
    h1 = _gemm_l0(t_num, n, d, bn)(s0, w0t)

    s1 = _make_spmm(t_num * d // C, n, ept, spmem_table=True)(
        h1.astype(jnp.bfloat16).reshape(-1, C), rc_p, vals_p)

    w1t = jnp.transpose(W[:, 1], (0, 2, 1))
    et2 = edge_types.reshape(n, 1).astype(jnp.int32)
    return _gemm_l1(t_num, n, d, bn)(s1, w1t, et2)


# speedup vs baseline: 4.4397x; 1.1315x over previous
import functools

import jax
import jax.numpy as jnp
from jax import lax
from jax.experimental import pallas as pl
from jax.experimental.pallas import tpu as pltpu
from jax.experimental.pallas import tpu_sc as plsc

NC = 2
NS = 16
LANES = 16
C = 64
BLK = 128


def _make_spmm(num_chunks: int, n: int, ept: int,
               spmem_table: bool = False, bf16_table: bool = False):
    bf16 = spmem_table or bf16_table
    assert num_chunks % NC == 0 and ept % (3 * BLK) == 0 and n % 8 == 0
    chunks_per_core = num_chunks // NC
    nblk = ept // BLK
    RPT = (n // 8 // NS) * 8
    REM = n - NS * RPT
    FL = 208
    assert RPT % FL == 0 and REM % 8 == 0 and REM < FL
    nfl = RPT // FL

    mesh = plsc.VectorSubcoreMesh(
        core_axis_name="c", subcore_axis_name="s",
        num_cores=NC, num_subcores=NS)

    @functools.partial(
        pl.kernel,
        out_type=jax.ShapeDtypeStruct((num_chunks, n, C), jnp.float32),
        mesh=mesh,
        compiler_params=pltpu.CompilerParams(
            use_tc_tiling_on_sc=False, needs_layout_passes=False),
        scratch_types=[
            pltpu.VMEM((ept,), jnp.int32),
            pltpu.VMEM((ept,), jnp.bfloat16),
            [pltpu.VMEM((BLK, C),
                        jnp.bfloat16 if bf16 else jnp.float32)
             for _ in range(3)],
            ([pltpu.VMEM((BLK, C), jnp.float32) for _ in range(3)]
             if bf16 else []),
            [pltpu.VMEM((BLK,), jnp.int32) for _ in range(3)],
            [pltpu.VMEM((BLK,), jnp.int32) for _ in range(3)],
            pltpu.VMEM((FL, C), jnp.float32),
            pltpu.VMEM_SHARED((n, C), jnp.float32),
            ([pltpu.VMEM_SHARED((n, C), jnp.bfloat16)]
             if spmem_table else []),
            [pltpu.SemaphoreType.DMA for _ in range(3)],
            [pltpu.SemaphoreType.DMA for _ in range(3)],
        ],
    )
    def spmm(h_hbm, rc_hbm, vals_hbm, out_hbm,
             rc_v, vals_v, gbuf, sbuf, idx_g, idx_s, zbuf, acc,
             tbl_l, gsem, ssem):
        tbl = tbl_l[0] if spmem_table else None
        src_of = (lambda p: sbuf[p]) if bf16 else (lambda p: gbuf[p])
        cid = lax.axis_index("c")
        sid = lax.axis_index("s")
        ebase = sid * ept
        pltpu.sync_copy(rc_hbm.at[pl.ds(ebase, ept)], rc_v)
        pltpu.sync_copy(vals_hbm.at[pl.ds(ebase, ept)], vals_v)

        zeros16 = jnp.zeros((LANES,), jnp.float32)

        def zrow(r, carry):
            for c8 in range(C // LANES):
                zbuf[r, pl.ds(c8 * LANES, LANES)] = zeros16
            return carry
        lax.fori_loop(0, FL, zrow, 0)

        def build_idx(i, joff, p):
            eoff = i * BLK
            for q in range(BLK // LANES):
                rcv = rc_v[pl.ds(eoff + q * LANES, LANES)]
                idx_g[p][pl.ds(q * LANES, LANES)] = (rcv & 16383) + joff
                idx_s[p][pl.ds(q * LANES, LANES)] = (
                    lax.shift_right_logical(rcv, 14))

        def widen(xb):
            xi = plsc.bitcast(xb, jnp.int32)
            lo = plsc.bitcast(lax.shift_left(xi, 16), jnp.float32)
            hi = plsc.bitcast(
                lax.bitwise_and(xi, jnp.int32(-65536)), jnp.float32)
            return lo, hi

        def scale(i, p):
            eoff = i * BLK
            for g2 in range(BLK // (2 * LANES)):
                vb = vals_v[pl.ds(eoff + 2 * g2 * LANES, 2 * LANES)]
                vpair = widen(vb)
                for half in range(2):
                    vv = vpair[half]
                    for u in range(LANES):
                        valv = jnp.full((LANES,), vv[u], jnp.float32)
                        row = 2 * g2 * LANES + half * LANES + u
                        if not bf16:
                            for k in range(C // LANES):
                                g = gbuf[p][row, k * LANES:(k + 1) * LANES]
                                gbuf[p][row, k * LANES:(k + 1) * LANES] = (
                                    g * valv)
                        else:
                            for h in range(C // (2 * LANES)):
                                lo, hi = widen(
                                    gbuf[p][row,
                                            2 * h * LANES:2 * (h + 1) * LANES])
                                c0 = 2 * h * LANES
                                sbuf[p][row, c0:c0 + LANES] = lo * valv
                                sbuf[p][row, c0 + LANES:c0 + 2 * LANES] = (
                                    hi * valv)

        gsrc = tbl if spmem_table else h_hbm

        def gather_start(i, p):
            pltpu.make_async_copy(gsrc.at[idx_g[p]], gbuf[p], gsem[p]).start()

        def gather_wait(p):
            pltpu.make_async_copy(gsrc.at[idx_g[p]], gbuf[p], gsem[p]).wait()

        def scatter_start(p):
            pltpu.async_copy(src_of(p), acc.at[idx_s[p]], ssem[p], add=True)

        def scatter_wait(p):
            pltpu.make_async_copy(src_of(p), acc.at[idx_s[p]],
                                  ssem[p]).wait()

        def chunk_body(jj, carry):
            j = cid + jj * NC
            for p in range(nfl):
                r0 = sid * RPT + p * FL
                pltpu.sync_copy(zbuf, acc.at[pl.ds(r0, FL)])
            if REM:
                @pl.when(sid == NS - 1)
                def _zero_tail():
                    pltpu.sync_copy(zbuf.at[pl.ds(0, REM)],
                                    acc.at[pl.ds(NS * RPT, REM)])
            plsc.subcore_barrier()

            joff = jnp.int32(0) if spmem_table else j * n

            if spmem_table:
                t0 = sid * RPT
                pltpu.sync_copy(h_hbm.at[pl.ds(j * n + t0, RPT)],
                                tbl.at[pl.ds(t0, RPT)])
                if REM:
                    @pl.when(sid == NS - 1)
                    def _tbl_tail():
                        pltpu.sync_copy(
                            h_hbm.at[pl.ds(j * n + NS * RPT, REM)],
                            tbl.at[pl.ds(NS * RPT, REM)])
                plsc.subcore_barrier()

            build_idx(0, joff, 0)
            gather_start(0, 0)
            build_idx(1, joff, 1)
            gather_start(1, 1)

            def eblk3(g, carry):
                for p in range(3):
                    i = 3 * g + p
                    q = (p + 2) % 3
                    gather_wait(p)
                    scale(i, p)
                    scatter_start(p)

                    @pl.when(i >= 1)
                    def _drain_prev():
                        scatter_wait(q)

                    @pl.when(i + 2 < nblk)
                    def _prefetch():
                        build_idx(i + 2, joff, q)
                        gather_start(i + 2, q)
                return carry
            lax.fori_loop(0, nblk // 3, eblk3, 0)
            scatter_wait((nblk - 1) % 3)
            plsc.subcore_barrier()

            r0 = sid * RPT
            pltpu.sync_copy(acc.at[pl.ds(r0, RPT)],
                            out_hbm.at[j].at[pl.ds(r0, RPT)])
            if REM:
                @pl.when(sid == NS - 1)
                def _flush_tail():
                    pltpu.sync_copy(acc.at[pl.ds(NS * RPT, REM)],
                                    out_hbm.at[j].at[pl.ds(NS * RPT, REM)])
            plsc.subcore_barrier()
            return carry
        lax.fori_loop(0, chunks_per_core, chunk_body, 0)

    return spmm


def _gemm_l0(t_num: int, n: int, d: int, bn: int):
    kin = d // C
    kout = t_num * d // C

    def body(s_ref, w_ref, out_ref):
        x = jnp.concatenate([s_ref[i] for i in range(kin)], axis=1)
        for t in range(t_num):
            y = lax.dot(x, w_ref[t], precision=lax.Precision.HIGHEST,
                        preferred_element_type=jnp.float32)
            y = jnp.maximum(y, 0.0)
            for u in range(kin):
                out_ref[t * kin + u] = y[:, u * C:(u + 1) * C]

    return pl.pallas_call(
        body,
        grid=(n // bn,),
        in_specs=[
            pl.BlockSpec((kin, bn, C), lambda i: (0, i, 0)),
            pl.BlockSpec((t_num, d, d), lambda i: (0, 0, 0)),
        ],
        out_specs=pl.BlockSpec((kout, bn, C), lambda i: (0, i, 0)),
        out_shape=jax.ShapeDtypeStruct((kout, n, C), jnp.float32),
    )


def _gemm_l1(t_num: int, n: int, d: int, bn: int):
    kin = d // C

    def body(s_ref, w_ref, et_ref, out_ref):
        et = et_ref[...]
        accum = jnp.zeros((bn, d), jnp.float32)
        for t in range(t_num):
            xt = jnp.concatenate(
                [s_ref[t * kin + u] for u in range(kin)], axis=1)
            y = lax.dot(xt, w_ref[t], precision=lax.Precision.HIGHEST,
                        preferred_element_type=jnp.float32)
            y = jnp.maximum(y, 0.0)
            accum = jnp.where(et == t, y, accum)
        out_ref[...] = accum

    return pl.pallas_call(
        body,
        grid=(n // bn,),
        in_specs=[
            pl.BlockSpec((t_num * kin, bn, C), lambda i: (0, i, 0)),
            pl.BlockSpec((t_num, d, d), lambda i: (0, 0, 0)),
            pl.BlockSpec((bn, 1), lambda i: (i, 0)),
        ],
        out_specs=pl.BlockSpec((bn, d), lambda i: (i, 0)),
        out_shape=jax.ShapeDtypeStruct((n, d), jnp.float32),
    )


def kernel(features, edge_types, adj_rows, adj_cols, adj_vals, W):
    n, d = features.shape
    e = adj_rows.shape[0]
    t_num = W.shape[0]
    bn = 1000

    ept = -(-e // (NS * 3 * BLK)) * 3 * BLK
    epad = NS * ept - e
    rc_p = jnp.pad((adj_rows.astype(jnp.int32) << 14)
                   | adj_cols.astype(jnp.int32), (0, epad))
    vals_p = jnp.pad(adj_vals, (0, epad)).astype(jnp.bfloat16)
    vals_p = vals_p.reshape(-1, 2, 16).swapaxes(1, 2).reshape(-1)

    import numpy as _np
    perm = _np.arange(d).reshape(d // 32, 2, 16).transpose(0, 2, 1).reshape(-1)
    f_flat = (features[:, perm].astype(jnp.bfloat16)
              .reshape(n, d // C, C).transpose(1, 0, 2).reshape(-1, C))
    s0 = _make_spmm(d // C, n, ept, bf16_table=True)(f_flat, rc_p, vals_p)

    w0t = jnp.transpose(W[:, 0], (0, 2, 1))
    w0t = w0t[:, :, perm]
    h1 = _gemm_l0(t_num, n, d, bn)(s0, w0t)

    s1 = _make_spmm(t_num * d // C, n, ept, spmem_table=True)(
        h1.astype(jnp.bfloat16).reshape(-1, C), rc_p, vals_p)

    w1t = jnp.transpose(W[:, 1], (0, 2, 1))
    et2 = edge_types.reshape(n, 1).astype(jnp.int32)
    return _gemm_l1(t_num, n, d, bn)(s1, w1t, et2)


# speedup vs baseline: 4.5147x; 1.0169x over previous
import functools

import jax
import jax.numpy as jnp
from jax import lax
from jax.experimental import pallas as pl
from jax.experimental.pallas import tpu as pltpu
from jax.experimental.pallas import tpu_sc as plsc

NC = 2
NS = 16
LANES = 16
C = 64
BLK = 128


def _make_spmm(num_chunks: int, n: int, ept: int,
               spmem_table: bool = False, bf16_table: bool = False):
    bf16 = spmem_table or bf16_table
    assert num_chunks % NC == 0 and ept % (3 * BLK) == 0 and n % 8 == 0
    chunks_per_core = num_chunks // NC
    nblk = ept // BLK
    RPT = (n // 8 // NS) * 8
    REM = n - NS * RPT
    FL = 208
    assert RPT % FL == 0 and REM % 8 == 0 and REM < FL
    nfl = RPT // FL

    mesh = plsc.VectorSubcoreMesh(
        core_axis_name="c", subcore_axis_name="s",
        num_cores=NC, num_subcores=NS)

    @functools.partial(
        pl.kernel,
        out_type=jax.ShapeDtypeStruct((num_chunks, n, C), jnp.float32),
        mesh=mesh,
        compiler_params=pltpu.CompilerParams(
            use_tc_tiling_on_sc=False, needs_layout_passes=False),
        scratch_types=[
            pltpu.VMEM((ept,), jnp.int32),
            pltpu.VMEM((ept,), jnp.bfloat16),
            [pltpu.VMEM((BLK, C),
                        jnp.bfloat16 if bf16 else jnp.float32)
             for _ in range(3)],
            ([pltpu.VMEM((BLK, C), jnp.float32) for _ in range(3)]
             if bf16 else []),
            [pltpu.VMEM((BLK,), jnp.int32) for _ in range(3)],
            [pltpu.VMEM((BLK,), jnp.int32) for _ in range(3)],
            pltpu.VMEM((FL, C), jnp.float32),
            pltpu.VMEM_SHARED((n, C), jnp.float32),
            ([pltpu.VMEM_SHARED((n, C), jnp.bfloat16)]
             if spmem_table else []),
            [pltpu.SemaphoreType.DMA for _ in range(3)],
            [pltpu.SemaphoreType.DMA for _ in range(3)],
        ],
    )
    def spmm(h_hbm, rc_hbm, vals_hbm, out_hbm,
             rc_v, vals_v, gbuf, sbuf, idx_g, idx_s, zbuf, acc,
             tbl_l, gsem, ssem):
        tbl = tbl_l[0] if spmem_table else None
        src_of = (lambda p: sbuf[p]) if bf16 else (lambda p: gbuf[p])
        cid = lax.axis_index("c")
        sid = lax.axis_index("s")
        ebase = sid * ept
        pltpu.sync_copy(rc_hbm.at[pl.ds(ebase, ept)], rc_v)
        pltpu.sync_copy(vals_hbm.at[pl.ds(ebase, ept)], vals_v)

        zeros16 = jnp.zeros((LANES,), jnp.float32)

        def zrow(r, carry):
            for c8 in range(C // LANES):
                zbuf[r, pl.ds(c8 * LANES, LANES)] = zeros16
            return carry
        lax.fori_loop(0, FL, zrow, 0)

        def build_idx(i, joff, p):
            eoff = i * BLK
            for q in range(BLK // LANES):
                rcv = rc_v[pl.ds(eoff + q * LANES, LANES)]
                idx_g[p][pl.ds(q * LANES, LANES)] = (rcv & 16383) + joff
                idx_s[p][pl.ds(q * LANES, LANES)] = (
                    lax.shift_right_logical(rcv, 14))

        def widen(xb):
            xi = plsc.bitcast(xb, jnp.int32)
            lo = plsc.bitcast(lax.shift_left(xi, 16), jnp.float32)
            hi = plsc.bitcast(
                lax.bitwise_and(xi, jnp.int32(-65536)), jnp.float32)
            return lo, hi

        def scale(i, p):
            eoff = i * BLK
            for g2 in range(BLK // (2 * LANES)):
                vb = vals_v[pl.ds(eoff + 2 * g2 * LANES, 2 * LANES)]
                vpair = widen(vb)
                for half in range(2):
                    vv = vpair[half]
                    for u in range(LANES):
                        valv = jnp.full((LANES,), vv[u], jnp.float32)
                        row = 2 * g2 * LANES + half * LANES + u
                        if not bf16:
                            for k in range(C // LANES):
                                g = gbuf[p][row, k * LANES:(k + 1) * LANES]
                                gbuf[p][row, k * LANES:(k + 1) * LANES] = (
                                    g * valv)
                        else:
                            for h in range(C // (2 * LANES)):
                                lo, hi = widen(
                                    gbuf[p][row,
                                            2 * h * LANES:2 * (h + 1) * LANES])
                                c0 = 2 * h * LANES
                                sbuf[p][row, c0:c0 + LANES] = lo * valv
                                sbuf[p][row, c0 + LANES:c0 + 2 * LANES] = (
                                    hi * valv)

        gsrc = tbl if spmem_table else h_hbm

        def gather_start(i, p):
            pltpu.make_async_copy(gsrc.at[idx_g[p]], gbuf[p], gsem[p]).start()

        def gather_wait(p):
            pltpu.make_async_copy(gsrc.at[idx_g[p]], gbuf[p], gsem[p]).wait()

        def scatter_start(p):
            pltpu.async_copy(src_of(p), acc.at[idx_s[p]], ssem[p], add=True)

        def scatter_wait(p):
            pltpu.make_async_copy(src_of(p), acc.at[idx_s[p]],
                                  ssem[p]).wait()

        def chunk_body(jj, carry):
            j = cid + jj * NC
            for p in range(nfl):
                r0 = sid * RPT + p * FL
                pltpu.sync_copy(zbuf, acc.at[pl.ds(r0, FL)])
            if REM:
                @pl.when(sid == NS - 1)
                def _zero_tail():
                    pltpu.sync_copy(zbuf.at[pl.ds(0, REM)],
                                    acc.at[pl.ds(NS * RPT, REM)])
            plsc.subcore_barrier()

            joff = jnp.int32(0) if spmem_table else j * n

            if spmem_table:
                t0 = sid * RPT
                pltpu.sync_copy(h_hbm.at[pl.ds(j * n + t0, RPT)],
                                tbl.at[pl.ds(t0, RPT)])
                if REM:
                    @pl.when(sid == NS - 1)
                    def _tbl_tail():
                        pltpu.sync_copy(
                            h_hbm.at[pl.ds(j * n + NS * RPT, REM)],
                            tbl.at[pl.ds(NS * RPT, REM)])
                plsc.subcore_barrier()

            build_idx(0, joff, 0)
            gather_start(0, 0)
            build_idx(1, joff, 1)
            gather_start(1, 1)

            def eblk3(g, carry):
                for p in range(3):
                    i = 3 * g + p
                    q = (p + 2) % 3
                    gather_wait(p)
                    scale(i, p)
                    scatter_start(p)

                    @pl.when(i >= 1)
                    def _drain_prev():
                        scatter_wait(q)

                    @pl.when(i + 2 < nblk)
                    def _prefetch():
                        build_idx(i + 2, joff, q)
                        gather_start(i + 2, q)
                return carry
            lax.fori_loop(0, nblk // 3, eblk3, 0)
            scatter_wait((nblk - 1) % 3)
            plsc.subcore_barrier()

            r0 = sid * RPT
            pltpu.sync_copy(acc.at[pl.ds(r0, RPT)],
                            out_hbm.at[j].at[pl.ds(r0, RPT)])
            if REM:
                @pl.when(sid == NS - 1)
                def _flush_tail():
                    pltpu.sync_copy(acc.at[pl.ds(NS * RPT, REM)],
                                    out_hbm.at[j].at[pl.ds(NS * RPT, REM)])
            plsc.subcore_barrier()
            return carry
        lax.fori_loop(0, chunks_per_core, chunk_body, 0)

    return spmm


def _gemm_l0(t_num: int, n: int, d: int, bn: int):
    kin = d // C
    kout = t_num * d // C

    def body(s_ref, w_ref, out_ref):
        x = jnp.concatenate([s_ref[i] for i in range(kin)], axis=1)
        for t in range(t_num):
            y = lax.dot(x, w_ref[t], precision=lax.Precision.DEFAULT,
                        preferred_element_type=jnp.float32)
            y = jnp.maximum(y, 0.0)
            for u in range(kin):
                out_ref[t * kin + u] = y[:, u * C:(u + 1) * C]

    return pl.pallas_call(
        body,
        grid=(n // bn,),
        in_specs=[
            pl.BlockSpec((kin, bn, C), lambda i: (0, i, 0)),
            pl.BlockSpec((t_num, d, d), lambda i: (0, 0, 0)),
        ],
        out_specs=pl.BlockSpec((kout, bn, C), lambda i: (0, i, 0)),
        out_shape=jax.ShapeDtypeStruct((kout, n, C), jnp.float32),
    )


def _gemm_l1(t_num: int, n: int, d: int, bn: int):
    kin = d // C

    def body(s_ref, w_ref, et_ref, out_ref):
        et = et_ref[...]
        accum = jnp.zeros((bn, d), jnp.float32)
        for t in range(t_num):
            xt = jnp.concatenate(
                [s_ref[t * kin + u] for u in range(kin)], axis=1)
            y = lax.dot(xt, w_ref[t], precision=lax.Precision.DEFAULT,
                        preferred_element_type=jnp.float32)
            y = jnp.maximum(y, 0.0)
            accum = jnp.where(et == t, y, accum)
        out_ref[...] = accum

    return pl.pallas_call(
        body,
        grid=(n // bn,),
        in_specs=[
            pl.BlockSpec((t_num * kin, bn, C), lambda i: (0, i, 0)),
            pl.BlockSpec((t_num, d, d), lambda i: (0, 0, 0)),
            pl.BlockSpec((bn, 1), lambda i: (i, 0)),
        ],
        out_specs=pl.BlockSpec((bn, d), lambda i: (i, 0)),
        out_shape=jax.ShapeDtypeStruct((n, d), jnp.float32),
    )


def kernel(features, edge_types, adj_rows, adj_cols, adj_vals, W):
    n, d = features.shape
    e = adj_rows.shape[0]
    t_num = W.shape[0]
    bn = 1000

    ept = -(-e // (NS * 3 * BLK)) * 3 * BLK
    epad = NS * ept - e
    rc_p = jnp.pad((adj_rows.astype(jnp.int32) << 14)
                   | adj_cols.astype(jnp.int32), (0, epad))
    vals_p = jnp.pad(adj_vals, (0, epad)).astype(jnp.bfloat16)
    vals_p = vals_p.reshape(-1, 2, 16).swapaxes(1, 2).reshape(-1)

    import numpy as _np
    perm = _np.arange(d).reshape(d // 32, 2, 16).transpose(0, 2, 1).reshape(-1)
    f_flat = (features[:, perm].astype(jnp.bfloat16)
              .reshape(n, d // C, C).transpose(1, 0, 2).reshape(-1, C))
    s0 = _make_spmm(d // C, n, ept, bf16_table=True)(f_flat, rc_p, vals_p)

    w0t = jnp.transpose(W[:, 0], (0, 2, 1))
    w0t = w0t[:, :, perm]
    h1 = _gemm_l0(t_num, n, d, bn)(s0, w0t)

    s1 = _make_spmm(t_num * d // C, n, ept, spmem_table=True)(
        h1.astype(jnp.bfloat16).reshape(-1, C), rc_p, vals_p)

    w1t = jnp.transpose(W[:, 1], (0, 2, 1))
    et2 = edge_types.reshape(n, 1).astype(jnp.int32)
    return _gemm_l1(t_num, n, d, bn)(s1, w1t, et2)
